# baseline (speedup 1.0000x reference)
import jax
import jax.numpy as jnp
from jax import lax
from jax.experimental import pallas as pl
from jax.experimental.pallas import tpu as pltpu
from jax.experimental.pallas import tpu_sc as plsc

N = 10000
NPAD = 10240
E = 320000
H = 128
HH = H // 2
C = 40
SLOPE = 0.2

NC = 2
NS = 16
L = 16
NW = NC * NS
EPT = E // NW
K = 80
NCH = EPT // K
GB = 4
SB = 3
RPT = NPAD // NS
ZR = 40
DEN_R = NPAD // L


def _sc_gat_body(hl_hbm, hr_hbm, asrc_hbm, adst_hbm, src_hbm, dst_hbm, z_hbm,
                 out_hbm, den_hbm,
                 asrc_v, adst_v, src2_v, dst2_v, ex_v, den_v,
                 rows_v, scaled_v, zbuf_v, out_sh, gsems, ssems):
    cid = lax.axis_index("c")
    sid = lax.axis_index("s")
    wid = sid * NC + cid

    pltpu.sync_copy(asrc_hbm, asrc_v)
    pltpu.sync_copy(adst_hbm, adst_v)
    pltpu.sync_copy(src_hbm.at[wid], src2_v)
    pltpu.sync_copy(dst_hbm.at[wid], dst2_v)
    pltpu.sync_copy(z_hbm, zbuf_v)

    for t in range(RPT // ZR):
        pltpu.sync_copy(zbuf_v, out_sh.at[pl.ds(sid * RPT + t * ZR, ZR)])

    zero16 = jnp.zeros((L,), jnp.float32)

    @pl.loop(0, DEN_R)
    def _(i):
        den_v[i, :] = zero16

    for half, h_hbm in ((0, hl_hbm), (1, hr_hbm)):
        plsc.subcore_barrier()

        for b in range(GB):
            pltpu.async_copy(h_hbm.at[src2_v.at[b]], rows_v.at[b], gsems.at[b])

        @pl.loop(0, NCH)
        def _(c):
            if True:
                b = lax.rem(c, GB)
                s_slot = lax.rem(c, SB)
                pltpu.make_async_copy(h_hbm.at[src2_v.at[c]],
                                      rows_v.at[b], gsems.at[b]).wait()

                @pl.when(c >= SB)
                def _():
                    pltpu.make_async_copy(scaled_v.at[s_slot],
                                          out_sh.at[dst2_v.at[c]],
                                          ssems.at[s_slot]).wait()

                for g in range(K // L):
                    s_idx = src2_v[c, pl.ds(g * L, L)]
                    d_idx = dst2_v[c, pl.ds(g * L, L)]
                    av = plsc.load_gather(
                        asrc_v, [lax.shift_right_logical(s_idx, 4),
                                 lax.bitwise_and(s_idx, L - 1)])
                    bv = plsc.load_gather(
                        adst_v, [lax.shift_right_logical(d_idx, 4),
                                 lax.bitwise_and(d_idx, L - 1)])
                    lg = av + bv
                    lg = jnp.where(lg >= 0.0, lg, lg * SLOPE)
                    ev = jnp.exp(lg)
                    if half == 0:
                        plsc.addupdate_scatter(
                            den_v,
                            [lax.shift_right_logical(d_idx, 4),
                             lax.bitwise_and(d_idx, L - 1)],
                            ev)
                    @plsc.parallel_loop(0, L, unroll=4)
                    def _(j2, g=g, ev=ev):
                        j = g * L + j2
                        attn = ev.at[jnp.broadcast_to(j2, (L,))].get(
                            mode="promise_in_bounds")
                        for q in range(HH // L):
                            scaled_v[s_slot, j, pl.ds(q * L, L)] = (
                                rows_v[b, j, pl.ds(q * L, L)] * attn)

                @pl.when(c + GB < NCH)
                def _():
                    pltpu.async_copy(h_hbm.at[src2_v.at[c + GB]],
                                     rows_v.at[b], gsems.at[b])

                pltpu.async_copy(scaled_v.at[s_slot], out_sh.at[dst2_v.at[c]],
                                 ssems.at[s_slot], add=True)

        for i in range(SB):
            c = NCH - SB + i
            pltpu.make_async_copy(scaled_v.at[c % SB],
                                  out_sh.at[dst2_v.at[c]],
                                  ssems.at[c % SB]).wait()

        if half == 0:
            pltpu.sync_copy(den_v, den_hbm.at[wid])

        plsc.subcore_barrier()

        for t in range(RPT // ZR):
            r0 = sid * RPT + t * ZR
            pltpu.sync_copy(out_sh.at[pl.ds(r0, ZR)], zbuf_v)
            pltpu.sync_copy(zbuf_v, out_hbm.at[half, cid, pl.ds(r0, ZR)])
            if half == 0:
                pltpu.sync_copy(z_hbm, zbuf_v)
                pltpu.sync_copy(zbuf_v, out_sh.at[pl.ds(r0, ZR)])


def _sc_gat_layer(hl, hr, a_src, a_dst, src2, dst2, zrows):
    a_src = a_src.reshape(DEN_R, L)
    a_dst = a_dst.reshape(DEN_R, L)
    mesh = plsc.VectorSubcoreMesh(core_axis_name="c", subcore_axis_name="s",
                                  num_cores=NC, num_subcores=NS)
    fn = pl.kernel(
        _sc_gat_body,
        out_type=(jax.ShapeDtypeStruct((2, NC, NPAD, HH), jnp.float32),
                  jax.ShapeDtypeStruct((NW, DEN_R, L), jnp.float32)),
        mesh=mesh,
        scratch_types=(
            pltpu.VMEM((DEN_R, L), jnp.float32),
            pltpu.VMEM((DEN_R, L), jnp.float32),
            pltpu.VMEM((NCH, K), jnp.int32),
            pltpu.VMEM((NCH, K), jnp.int32),
            pltpu.VMEM((K // L, L), jnp.float32),
            pltpu.VMEM((DEN_R, L), jnp.float32),
            pltpu.VMEM((GB, K, HH), jnp.float32),
            pltpu.VMEM((SB, K, HH), jnp.float32),
            pltpu.VMEM((ZR, HH), jnp.float32),
            pltpu.VMEM_SHARED((NPAD, HH), jnp.float32),
            pltpu.SemaphoreType.DMA((GB,)),
            pltpu.SemaphoreType.DMA((SB,)),
        ),
        compiler_params=pltpu.CompilerParams(needs_layout_passes=False,
                                             use_tc_tiling_on_sc=False),
    )
    return fn(hl, hr, a_src, a_dst, src2, dst2, zrows)


def _tc_in_body(x_ref, w_ref, as_ref, ad_ref,
                hl_ref, hr_ref, als_ref, ald_ref):
    h = jnp.dot(x_ref[...], w_ref[...], preferred_element_type=jnp.float32)
    hl_ref[...] = h[:, :HH]
    hr_ref[...] = h[:, HH:]
    als_ref[...] = jnp.sum(h * as_ref[...], axis=1)
    ald_ref[...] = jnp.sum(h * ad_ref[...], axis=1)


def _tc_in(x, w, a_s, a_d):
    return pl.pallas_call(
        _tc_in_body,
        out_shape=(jax.ShapeDtypeStruct((NPAD, HH), jnp.float32),
                   jax.ShapeDtypeStruct((NPAD, HH), jnp.float32),
                   jax.ShapeDtypeStruct((NPAD,), jnp.float32),
                   jax.ShapeDtypeStruct((NPAD,), jnp.float32)),
    )(x, w, a_s, a_d)


def _combine(parts_ref, den_ref):
    agg = jnp.concatenate(
        [parts_ref[0, 0] + parts_ref[0, 1], parts_ref[1, 0] + parts_ref[1, 1]],
        axis=1)
    den = jnp.sum(den_ref[...], axis=0)
    return agg / (den[:, None] + 1e-16)


def _tc_mid_body(parts_ref, den_ref, bprev_ref, w_ref, as_ref, ad_ref,
                 hl_ref, hr_ref, als_ref, ald_ref):
    agg = _combine(parts_ref, den_ref) + bprev_ref[...]
    xin = jnp.where(agg > 0.0, agg, jnp.exp(agg) - 1.0)
    h = jnp.dot(xin, w_ref[...], preferred_element_type=jnp.float32)
    hl_ref[...] = h[:, :HH]
    hr_ref[...] = h[:, HH:]
    als_ref[...] = jnp.sum(h * as_ref[...], axis=1)
    ald_ref[...] = jnp.sum(h * ad_ref[...], axis=1)


def _tc_mid(parts, den, b_prev, w, a_s, a_d):
    return pl.pallas_call(
        _tc_mid_body,
        out_shape=(jax.ShapeDtypeStruct((NPAD, HH), jnp.float32),
                   jax.ShapeDtypeStruct((NPAD, HH), jnp.float32),
                   jax.ShapeDtypeStruct((NPAD,), jnp.float32),
                   jax.ShapeDtypeStruct((NPAD,), jnp.float32)),
    )(parts, den, b_prev, w, a_s, a_d)


def _tc_out_body(parts_ref, den_ref, b3_ref, wp_ref, bp_ref, wr_ref, br_ref,
                 y_ref):
    agg = _combine(parts_ref, den_ref) + b3_ref[...]
    hp = jnp.dot(agg, wp_ref[...], preferred_element_type=jnp.float32)
    hp = jnp.maximum(hp + bp_ref[...], 0.0)
    y_ref[...] = jnp.dot(hp, wr_ref[...],
                         preferred_element_type=jnp.float32) + br_ref[...]


def _tc_out(parts, den, b3, wp, bp, wr, br):
    return pl.pallas_call(
        _tc_out_body,
        out_shape=jax.ShapeDtypeStruct((NPAD, H), jnp.float32),
    )(parts, den, b3, wp, bp, wr, br)


@jax.jit
def kernel(x, edge_index, W1, a1s, a1d, b1, W2, a2s, a2d, b2,
           W3, a3s, a3d, b3, Wp, bp, Wr, br):
    src2 = edge_index[0].reshape(NW, NCH, K)
    dst2 = edge_index[1].reshape(NW, NCH, K)
    x_pad = jnp.pad(x, ((0, NPAD - N), (0, 0)))
    zrows = jnp.zeros((ZR, HH), jnp.float32)

    h1l, h1r, s1, d1 = _tc_in(x_pad, W1, a1s.reshape(1, H), a1d.reshape(1, H))
    out1, den1 = _sc_gat_layer(h1l, h1r, s1, d1, src2, dst2, zrows)
    den1 = den1.reshape(NW, NPAD)

    h2l, h2r, s2, d2 = _tc_mid(out1, den1, b1.reshape(1, H),
                               W2, a2s.reshape(1, H), a2d.reshape(1, H))
    out2, den2 = _sc_gat_layer(h2l, h2r, s2, d2, src2, dst2, zrows)
    den2 = den2.reshape(NW, NPAD)

    h3l, h3r, s3, d3 = _tc_mid(out2, den2, b2.reshape(1, H),
                               W3, a3s.reshape(1, H), a3d.reshape(1, H))
    out3, den3 = _sc_gat_layer(h3l, h3r, s3, d3, src2, dst2, zrows)
    den3 = den3.reshape(NW, NPAD)

    y = _tc_out(out3, den3, b3.reshape(1, H),
                Wp, bp.reshape(1, H),
                jnp.pad(Wr, ((0, 0), (0, H - C))),
                jnp.pad(br, (0, H - C)).reshape(1, H))
    return y[:N, :C]


# speedup vs baseline: 1.0082x; 1.0082x over previous
import jax
import jax.numpy as jnp
from jax import lax
from jax.experimental import pallas as pl
from jax.experimental.pallas import tpu as pltpu
from jax.experimental.pallas import tpu_sc as plsc

N = 10000
NPAD = 10240
E = 320000
H = 128
HH = H // 2
C = 40
SLOPE = 0.2

NC = 2
NS = 16
L = 16
NW = NC * NS
EPT = E // NW
K = 80
NCH = EPT // K
GB = 5
SB = 2
RPT = NPAD // NS
ZR = 40
DEN_R = NPAD // L


def _sc_gat_body(hl_hbm, hr_hbm, asrc_hbm, adst_hbm, src_hbm, dst_hbm, z_hbm,
                 out_hbm, den_hbm,
                 asrc_v, adst_v, src2_v, dst2_v, ex_v, den_v,
                 rows_v, scaled_v, zbuf_v, out_sh, gsems, ssems):
    cid = lax.axis_index("c")
    sid = lax.axis_index("s")
    wid = sid * NC + cid

    pltpu.sync_copy(asrc_hbm, asrc_v)
    pltpu.sync_copy(adst_hbm, adst_v)
    pltpu.sync_copy(src_hbm.at[wid], src2_v)
    pltpu.sync_copy(dst_hbm.at[wid], dst2_v)
    pltpu.sync_copy(z_hbm, zbuf_v)

    for t in range(RPT // ZR):
        pltpu.sync_copy(zbuf_v, out_sh.at[pl.ds(sid * RPT + t * ZR, ZR)])

    zero16 = jnp.zeros((L,), jnp.float32)

    @pl.loop(0, DEN_R)
    def _(i):
        den_v[i, :] = zero16

    for half, h_hbm in ((0, hl_hbm), (1, hr_hbm)):
        plsc.subcore_barrier()

        for b in range(GB):
            pltpu.async_copy(h_hbm.at[src2_v.at[b]], rows_v.at[b], gsems.at[b])

        @pl.loop(0, NCH)
        def _(c):
            if True:
                b = lax.rem(c, GB)
                s_slot = lax.bitwise_and(c, 1)
                pltpu.make_async_copy(h_hbm.at[src2_v.at[c]],
                                      rows_v.at[b], gsems.at[b]).wait()

                @pl.when(c >= SB)
                def _():
                    pltpu.make_async_copy(scaled_v.at[s_slot],
                                          out_sh.at[dst2_v.at[c]],
                                          ssems.at[s_slot]).wait()

                for g in range(K // L):
                    s_idx = src2_v[c, pl.ds(g * L, L)]
                    d_idx = dst2_v[c, pl.ds(g * L, L)]
                    av = plsc.load_gather(
                        asrc_v, [lax.shift_right_logical(s_idx, 4),
                                 lax.bitwise_and(s_idx, L - 1)])
                    bv = plsc.load_gather(
                        adst_v, [lax.shift_right_logical(d_idx, 4),
                                 lax.bitwise_and(d_idx, L - 1)])
                    lg = av + bv
                    lg = jnp.where(lg >= 0.0, lg, lg * SLOPE)
                    ev = jnp.exp(lg)
                    if half == 0:
                        plsc.addupdate_scatter(
                            den_v,
                            [lax.shift_right_logical(d_idx, 4),
                             lax.bitwise_and(d_idx, L - 1)],
                            ev)
                    @plsc.parallel_loop(0, L, unroll=4)
                    def _(j2, g=g, ev=ev):
                        j = g * L + j2
                        attn = ev.at[jnp.broadcast_to(j2, (L,))].get(
                            mode="promise_in_bounds")
                        for q in range(HH // L):
                            scaled_v[s_slot, j, pl.ds(q * L, L)] = (
                                rows_v[b, j, pl.ds(q * L, L)] * attn)

                @pl.when(c + GB < NCH)
                def _():
                    pltpu.async_copy(h_hbm.at[src2_v.at[c + GB]],
                                     rows_v.at[b], gsems.at[b])

                pltpu.async_copy(scaled_v.at[s_slot], out_sh.at[dst2_v.at[c]],
                                 ssems.at[s_slot], add=True)

        for i in range(SB):
            c = NCH - SB + i
            pltpu.make_async_copy(scaled_v.at[c % SB],
                                  out_sh.at[dst2_v.at[c]],
                                  ssems.at[c % SB]).wait()

        if half == 0:
            pltpu.sync_copy(den_v, den_hbm.at[wid])

        plsc.subcore_barrier()

        for t in range(RPT // ZR):
            r0 = sid * RPT + t * ZR
            pltpu.sync_copy(out_sh.at[pl.ds(r0, ZR)], zbuf_v)
            pltpu.sync_copy(zbuf_v, out_hbm.at[half, cid, pl.ds(r0, ZR)])
            if half == 0:
                pltpu.sync_copy(z_hbm, zbuf_v)
                pltpu.sync_copy(zbuf_v, out_sh.at[pl.ds(r0, ZR)])


def _sc_gat_layer(hl, hr, a_src, a_dst, src2, dst2, zrows):
    a_src = a_src.reshape(DEN_R, L)
    a_dst = a_dst.reshape(DEN_R, L)
    mesh = plsc.VectorSubcoreMesh(core_axis_name="c", subcore_axis_name="s",
                                  num_cores=NC, num_subcores=NS)
    fn = pl.kernel(
        _sc_gat_body,
        out_type=(jax.ShapeDtypeStruct((2, NC, NPAD, HH), jnp.float32),
                  jax.ShapeDtypeStruct((NW, DEN_R, L), jnp.float32)),
        mesh=mesh,
        scratch_types=(
            pltpu.VMEM((DEN_R, L), jnp.float32),
            pltpu.VMEM((DEN_R, L), jnp.float32),
            pltpu.VMEM((NCH, K), jnp.int32),
            pltpu.VMEM((NCH, K), jnp.int32),
            pltpu.VMEM((K // L, L), jnp.float32),
            pltpu.VMEM((DEN_R, L), jnp.float32),
            pltpu.VMEM((GB, K, HH), jnp.float32),
            pltpu.VMEM((SB, K, HH), jnp.float32),
            pltpu.VMEM((ZR, HH), jnp.float32),
            pltpu.VMEM_SHARED((NPAD, HH), jnp.float32),
            pltpu.SemaphoreType.DMA((GB,)),
            pltpu.SemaphoreType.DMA((SB,)),
        ),
        compiler_params=pltpu.CompilerParams(needs_layout_passes=False,
                                             use_tc_tiling_on_sc=False),
    )
    return fn(hl, hr, a_src, a_dst, src2, dst2, zrows)


def _tc_in_body(x_ref, w_ref, as_ref, ad_ref,
                hl_ref, hr_ref, als_ref, ald_ref):
    h = jnp.dot(x_ref[...], w_ref[...], preferred_element_type=jnp.float32)
    hl_ref[...] = h[:, :HH]
    hr_ref[...] = h[:, HH:]
    als_ref[...] = jnp.sum(h * as_ref[...], axis=1)
    ald_ref[...] = jnp.sum(h * ad_ref[...], axis=1)


def _tc_in(x, w, a_s, a_d):
    return pl.pallas_call(
        _tc_in_body,
        out_shape=(jax.ShapeDtypeStruct((NPAD, HH), jnp.float32),
                   jax.ShapeDtypeStruct((NPAD, HH), jnp.float32),
                   jax.ShapeDtypeStruct((NPAD,), jnp.float32),
                   jax.ShapeDtypeStruct((NPAD,), jnp.float32)),
    )(x, w, a_s, a_d)


def _combine(parts_ref, den_ref):
    agg = jnp.concatenate(
        [parts_ref[0, 0] + parts_ref[0, 1], parts_ref[1, 0] + parts_ref[1, 1]],
        axis=1)
    den = jnp.sum(den_ref[...], axis=0)
    return agg / (den[:, None] + 1e-16)


def _tc_mid_body(parts_ref, den_ref, bprev_ref, w_ref, as_ref, ad_ref,
                 hl_ref, hr_ref, als_ref, ald_ref):
    agg = _combine(parts_ref, den_ref) + bprev_ref[...]
    xin = jnp.where(agg > 0.0, agg, jnp.exp(agg) - 1.0)
    h = jnp.dot(xin, w_ref[...], preferred_element_type=jnp.float32)
    hl_ref[...] = h[:, :HH]
    hr_ref[...] = h[:, HH:]
    als_ref[...] = jnp.sum(h * as_ref[...], axis=1)
    ald_ref[...] = jnp.sum(h * ad_ref[...], axis=1)


def _tc_mid(parts, den, b_prev, w, a_s, a_d):
    return pl.pallas_call(
        _tc_mid_body,
        out_shape=(jax.ShapeDtypeStruct((NPAD, HH), jnp.float32),
                   jax.ShapeDtypeStruct((NPAD, HH), jnp.float32),
                   jax.ShapeDtypeStruct((NPAD,), jnp.float32),
                   jax.ShapeDtypeStruct((NPAD,), jnp.float32)),
    )(parts, den, b_prev, w, a_s, a_d)


def _tc_out_body(parts_ref, den_ref, b3_ref, wp_ref, bp_ref, wr_ref, br_ref,
                 y_ref):
    agg = _combine(parts_ref, den_ref) + b3_ref[...]
    hp = jnp.dot(agg, wp_ref[...], preferred_element_type=jnp.float32)
    hp = jnp.maximum(hp + bp_ref[...], 0.0)
    y_ref[...] = jnp.dot(hp, wr_ref[...],
                         preferred_element_type=jnp.float32) + br_ref[...]


def _tc_out(parts, den, b3, wp, bp, wr, br):
    return pl.pallas_call(
        _tc_out_body,
        out_shape=jax.ShapeDtypeStruct((NPAD, H), jnp.float32),
    )(parts, den, b3, wp, bp, wr, br)


@jax.jit
def kernel(x, edge_index, W1, a1s, a1d, b1, W2, a2s, a2d, b2,
           W3, a3s, a3d, b3, Wp, bp, Wr, br):
    src2 = edge_index[0].reshape(NW, NCH, K)
    dst2 = edge_index[1].reshape(NW, NCH, K)
    x_pad = jnp.pad(x, ((0, NPAD - N), (0, 0)))
    zrows = jnp.zeros((ZR, HH), jnp.float32)

    h1l, h1r, s1, d1 = _tc_in(x_pad, W1, a1s.reshape(1, H), a1d.reshape(1, H))
    out1, den1 = _sc_gat_layer(h1l, h1r, s1, d1, src2, dst2, zrows)
    den1 = den1.reshape(NW, NPAD)

    h2l, h2r, s2, d2 = _tc_mid(out1, den1, b1.reshape(1, H),
                               W2, a2s.reshape(1, H), a2d.reshape(1, H))
    out2, den2 = _sc_gat_layer(h2l, h2r, s2, d2, src2, dst2, zrows)
    den2 = den2.reshape(NW, NPAD)

    h3l, h3r, s3, d3 = _tc_mid(out2, den2, b2.reshape(1, H),
                               W3, a3s.reshape(1, H), a3d.reshape(1, H))
    out3, den3 = _sc_gat_layer(h3l, h3r, s3, d3, src2, dst2, zrows)
    den3 = den3.reshape(NW, NPAD)

    y = _tc_out(out3, den3, b3.reshape(1, H),
                Wp, bp.reshape(1, H),
                jnp.pad(Wr, ((0, 0), (0, H - C))),
                jnp.pad(br, (0, H - C)).reshape(1, H))
    return y[:N, :C]
